# baseline (speedup 1.0000x reference)
import functools

import jax
import jax.numpy as jnp
from jax import lax
from jax.experimental import pallas as pl
from jax.experimental.pallas import tpu as pltpu
from jax.experimental.pallas import tpu_sc as plsc

N = 10000
E = 320000
D = 128
HD = D // 2
NC = 2
NS = 16
K = 125
NB = 160
EPT = NB * K
NPAD = 10112
RPT = NPAD // NS
CW = 16


def _sc_aggregate(x2, src3, src3p, dst3, with_counts):
    mesh = plsc.VectorSubcoreMesh(core_axis_name="c", subcore_axis_name="s")

    @functools.partial(
        pl.kernel,
        out_type=[
            jax.ShapeDtypeStruct((NC, NPAD, HD), jnp.float32),
            jax.ShapeDtypeStruct((NPAD, CW), jnp.float32),
        ],
        mesh=mesh,
        compiler_params=pltpu.CompilerParams(use_tc_tiling_on_sc=False),
        scratch_types=[
            pltpu.VMEM((NB, K), jnp.int32),
            pltpu.VMEM((NB, K), jnp.int32),
            [pltpu.VMEM((K, HD), jnp.float32) for _ in range(4)],
            pltpu.VMEM((K, CW), jnp.float32),
            pltpu.VMEM((K, CW), jnp.float32),
            pltpu.VMEM_SHARED((NPAD, HD), jnp.float32),
            pltpu.VMEM_SHARED((NPAD, CW), jnp.float32),
            [pltpu.SemaphoreType.DMA for _ in range(4)],
            [pltpu.SemaphoreType.DMA for _ in range(4)],
            [pltpu.SemaphoreType.DMA for _ in range(2)],
        ],
    )
    def agg_kernel(x_hbm, src_hbm, srcp_hbm, dst_hbm, out_hbm, outc_hbm,
                   srcv, dstv, rows, ones, zcnt, acc_s, cnt_s, gs, ss, cs):
        c = lax.axis_index("c")
        s = lax.axis_index("s")

        def fill_zrow(i, _):
            for j in range(HD // 16):
                rows[0][i, pl.ds(j * 16, 16)] = jnp.zeros((16,),
                                                          jnp.float32)
            return 0
        lax.fori_loop(0, K, fill_zrow, 0)

        def fill_zcnt(i, _):
            zcnt[i, :] = jnp.zeros((16,), jnp.float32)
            if with_counts:
                ones[i, :] = jnp.ones((16,), jnp.float32)
            return 0
        lax.fori_loop(0, K, fill_zcnt, 0)

        base = s * RPT
        nz = RPT // K
        tail = RPT - nz * K
        for z in range(nz):
            pltpu.sync_copy(rows[0], acc_s.at[pl.ds(base + z * K, K)])
        pltpu.sync_copy(rows[0].at[pl.ds(0, tail)],
                        acc_s.at[pl.ds(base + nz * K, tail)])
        if with_counts:
            @pl.when(c == 0)
            def _():
                for z in range(nz):
                    pltpu.sync_copy(zcnt, cnt_s.at[pl.ds(base + z * K, K)])
                pltpu.sync_copy(zcnt.at[pl.ds(0, tail)],
                                cnt_s.at[pl.ds(base + nz * K, tail)])

        @pl.when(c == 0)
        def _():
            pltpu.sync_copy(src_hbm.at[s], srcv)

        @pl.when(c == 1)
        def _():
            pltpu.sync_copy(srcp_hbm.at[s], srcv)

        pltpu.sync_copy(dst_hbm.at[s], dstv)

        plsc.subcore_barrier()

        def g_start(i, b):
            pltpu.async_copy(x_hbm.at[srcv.at[i]], rows[b], gs[b])

        def g_wait(b):
            pltpu.make_async_copy(x_hbm.at[srcv.at[0]], rows[b],
                                  gs[b]).wait()

        def s_start(i, b):
            pltpu.async_copy(rows[b], acc_s.at[dstv.at[i]], ss[b], add=True)

        def s_wait(b):
            pltpu.make_async_copy(rows[b], acc_s.at[dstv.at[0]],
                                  ss[b]).wait()

        def cnt_fire(wait_prev, i, b):
            @pl.when(c == 0)
            def _():
                if wait_prev is True:
                    pltpu.make_async_copy(ones, cnt_s.at[dstv.at[0]],
                                          cs[b]).wait()
                else:
                    @pl.when(wait_prev)
                    def _():
                        pltpu.make_async_copy(ones, cnt_s.at[dstv.at[0]],
                                              cs[b]).wait()
                pltpu.async_copy(ones, cnt_s.at[dstv.at[i]], cs[b],
                                 add=True)

        g_start(0, 0)
        g_start(1, 1)
        g_start(2, 2)

        NSW4 = NB // 4

        def body(j, _):
            for b in range(4):
                i = 4 * j + b
                g_wait(b)
                s_start(i, b)
                pb = (b + 3) % 4
                if b == 0:
                    @pl.when(j > 0)
                    def _():
                        s_wait(pb)
                    g_start(i + 3, pb)
                else:
                    s_wait(pb)

                    @pl.when(j < NSW4 - 1)
                    def _():
                        g_start(i + 3, pb)
                if with_counts:
                    cnt_fire((j > 0) if b < 2 else True, i, b % 2)
            return 0
        lax.fori_loop(0, NSW4, body, 0)

        s_wait(3)
        if with_counts:
            @pl.when(c == 0)
            def _():
                pltpu.make_async_copy(ones, cnt_s.at[dstv.at[0]],
                                      cs[0]).wait()
                pltpu.make_async_copy(ones, cnt_s.at[dstv.at[0]],
                                      cs[1]).wait()

        plsc.subcore_barrier()
        pltpu.sync_copy(acc_s.at[pl.ds(base, RPT)],
                        out_hbm.at[c, pl.ds(base, RPT)])
        if with_counts:
            @pl.when(c == 0)
            def _():
                pltpu.sync_copy(cnt_s.at[pl.ds(base, RPT)],
                                outc_hbm.at[pl.ds(base, RPT)])

    return agg_kernel(x2, src3, src3p, dst3)


def _split_stack(h):
    return jnp.concatenate([h[:, :HD], h[:, HD:]], axis=0)


BR = 1000


def _tc_self_path(x, W, b):
    def body(x_ref, W_ref, b_ref, o_ref):
        o_ref[...] = (jnp.dot(x_ref[...], W_ref[...],
                              preferred_element_type=jnp.float32)
                      + b_ref[...])

    return pl.pallas_call(
        body,
        grid=(N // BR,),
        in_specs=[
            pl.BlockSpec((BR, D), lambda i: (i, 0)),
            pl.BlockSpec((D, D), lambda i: (0, 0)),
            pl.BlockSpec((1, D), lambda i: (0, 0)),
        ],
        out_specs=pl.BlockSpec((BR, D), lambda i: (i, 0)),
        out_shape=jax.ShapeDtypeStruct((N, D), jnp.float32),
    )(x, W, b)


def _tc_layer1(S, C, xr, Wl1, gamma, beta):
    def body(S_ref, C_ref, xr_ref, Wl1_ref, g_ref, b_ref,
             h2_ref, h2s_ref):
        inv = 1.0 / jnp.maximum(C_ref[:N, 0:1], 1.0)
        aggL = S_ref[0, :N, :] * inv
        aggR = S_ref[1, :N, :] * inv
        h = (jnp.dot(aggL, Wl1_ref[:HD, :],
                     preferred_element_type=jnp.float32)
             + jnp.dot(aggR, Wl1_ref[HD:, :],
                       preferred_element_type=jnp.float32)
             + xr_ref[...])
        mu = jnp.mean(h, axis=0, keepdims=True)
        var = jnp.mean((h - mu) * (h - mu), axis=0, keepdims=True)
        hn = (h - mu) / jnp.sqrt(var + 1e-5) * g_ref[...] + b_ref[...]
        h2 = jnp.where(hn >= 0, hn, 0.01 * hn)
        h2_ref[...] = h2
        h2s_ref[0] = h2[:, :HD]
        h2s_ref[1] = h2[:, HD:]

    return pl.pallas_call(
        body,
        out_shape=[
            jax.ShapeDtypeStruct((N, D), jnp.float32),
            jax.ShapeDtypeStruct((2, N, HD), jnp.float32),
        ],
    )(S, C, xr, Wl1, gamma, beta)


def _tc_layer2(S2, C, r2, Wl2):
    def body(S_ref, C_ref, r2_ref, Wl2_ref, out_ref):
        inv = 1.0 / jnp.maximum(C_ref[:, 0:1], 1.0)
        aggL = S_ref[0] * inv
        aggR = S_ref[1] * inv
        out_ref[...] = (jnp.dot(aggL, Wl2_ref[:HD, :],
                                preferred_element_type=jnp.float32)
                        + jnp.dot(aggR, Wl2_ref[HD:, :],
                                  preferred_element_type=jnp.float32)
                        + r2_ref[...])

    return pl.pallas_call(
        body,
        grid=(N // BR,),
        in_specs=[
            pl.BlockSpec((2, BR, HD), lambda i: (0, i, 0)),
            pl.BlockSpec((BR, CW), lambda i: (i, 0)),
            pl.BlockSpec((BR, D), lambda i: (i, 0)),
            pl.BlockSpec((D, D), lambda i: (0, 0)),
        ],
        out_specs=pl.BlockSpec((BR, D), lambda i: (i, 0)),
        out_shape=jax.ShapeDtypeStruct((N, D), jnp.float32),
    )(S2, C, r2, Wl2)


def kernel(x, edge_index, Wl1, bl1, Wr1, gamma, beta, Wl2, bl2, Wr2):
    src3 = edge_index[0].astype(jnp.int32).reshape(NS, NB, K)
    src3p = src3 + N
    dst3 = edge_index[1].astype(jnp.int32).reshape(NS, NB, K)
    bl1r = bl1.reshape(1, D)
    bl2r = bl2.reshape(1, D)
    gr = gamma.reshape(1, D)
    br = beta.reshape(1, D)

    S1, C = _sc_aggregate(_split_stack(x), src3, src3p, dst3,
                          with_counts=True)
    xr = _tc_self_path(x, Wr1, bl1r)
    h2, h2s = _tc_layer1(S1, C, xr, Wl1, gr, br)
    S2, _ = _sc_aggregate(h2s.reshape(2 * N, HD), src3, src3p, dst3,
                          with_counts=False)
    r2 = _tc_self_path(h2, Wr2, bl2r)
    return _tc_layer2(S2, C, r2, Wl2)


# speedup vs baseline: 1.0081x; 1.0081x over previous
import functools

import jax
import jax.numpy as jnp
from jax import lax
from jax.experimental import pallas as pl
from jax.experimental.pallas import tpu as pltpu
from jax.experimental.pallas import tpu_sc as plsc

N = 10000
E = 320000
D = 128
HD = D // 2
NC = 2
NS = 16
K = 125
NB = 160
EPT = NB * K
NPAD = 10112
RPT = NPAD // NS
CW = 16


def _sc_aggregate(x2, src3, src3p, dst3, with_counts):
    mesh = plsc.VectorSubcoreMesh(core_axis_name="c", subcore_axis_name="s")

    @functools.partial(
        pl.kernel,
        out_type=[
            jax.ShapeDtypeStruct((NC, NPAD, HD), jnp.float32),
            jax.ShapeDtypeStruct((NPAD, CW), jnp.float32),
        ],
        mesh=mesh,
        compiler_params=pltpu.CompilerParams(use_tc_tiling_on_sc=False),
        scratch_types=[
            pltpu.VMEM((NB, K), jnp.int32),
            pltpu.VMEM((NB, K), jnp.int32),
            [pltpu.VMEM((K, HD), jnp.float32) for _ in range(4)],
            pltpu.VMEM((K, CW), jnp.float32),
            pltpu.VMEM((K, CW), jnp.float32),
            pltpu.VMEM_SHARED((NPAD, HD), jnp.float32),
            pltpu.VMEM_SHARED((NPAD, CW), jnp.float32),
            [pltpu.SemaphoreType.DMA for _ in range(4)],
            [pltpu.SemaphoreType.DMA for _ in range(4)],
            [pltpu.SemaphoreType.DMA for _ in range(2)],
        ],
    )
    def agg_kernel(x_hbm, src_hbm, srcp_hbm, dst_hbm, out_hbm, outc_hbm,
                   srcv, dstv, rows, ones, zcnt, acc_s, cnt_s, gs, ss, cs):
        c = lax.axis_index("c")
        s = lax.axis_index("s")

        def fill_zrow(i, _):
            for j in range(HD // 16):
                rows[0][i, pl.ds(j * 16, 16)] = jnp.zeros((16,),
                                                          jnp.float32)
            return 0
        lax.fori_loop(0, K, fill_zrow, 0)

        def fill_zcnt(i, _):
            zcnt[i, :] = jnp.zeros((16,), jnp.float32)
            if with_counts:
                ones[i, :] = jnp.ones((16,), jnp.float32)
            return 0
        lax.fori_loop(0, K, fill_zcnt, 0)

        base = s * RPT
        nz = RPT // K
        tail = RPT - nz * K
        for z in range(nz):
            pltpu.sync_copy(rows[0], acc_s.at[pl.ds(base + z * K, K)])
        pltpu.sync_copy(rows[0].at[pl.ds(0, tail)],
                        acc_s.at[pl.ds(base + nz * K, tail)])
        if with_counts:
            @pl.when(c == 0)
            def _():
                for z in range(nz):
                    pltpu.sync_copy(zcnt, cnt_s.at[pl.ds(base + z * K, K)])
                pltpu.sync_copy(zcnt.at[pl.ds(0, tail)],
                                cnt_s.at[pl.ds(base + nz * K, tail)])

        @pl.when(c == 0)
        def _():
            pltpu.sync_copy(src_hbm.at[s], srcv)

        @pl.when(c == 1)
        def _():
            pltpu.sync_copy(srcp_hbm.at[s], srcv)

        pltpu.sync_copy(dst_hbm.at[s], dstv)

        plsc.subcore_barrier()

        def g_start(i, b):
            pltpu.async_copy(x_hbm.at[srcv.at[i]], rows[b], gs[b])

        def g_wait(b):
            pltpu.make_async_copy(x_hbm.at[srcv.at[0]], rows[b],
                                  gs[b]).wait()

        def s_start(i, b):
            pltpu.async_copy(rows[b], acc_s.at[dstv.at[i]], ss[b], add=True)

        def s_wait(b):
            pltpu.make_async_copy(rows[b], acc_s.at[dstv.at[0]],
                                  ss[b]).wait()

        def cnt_fire(wait_prev, i, b):
            @pl.when(c == 0)
            def _():
                if wait_prev is True:
                    pltpu.make_async_copy(ones, cnt_s.at[dstv.at[0]],
                                          cs[b]).wait()
                else:
                    @pl.when(wait_prev)
                    def _():
                        pltpu.make_async_copy(ones, cnt_s.at[dstv.at[0]],
                                              cs[b]).wait()
                pltpu.async_copy(ones, cnt_s.at[dstv.at[i]], cs[b],
                                 add=True)

        g_start(0, 0)
        g_start(1, 1)
        g_start(2, 2)

        NSW4 = NB // 4

        def body(j, _):
            for b in range(4):
                i = 4 * j + b
                g_wait(b)
                s_start(i, b)
                pb = (b + 3) % 4
                if b == 0:
                    @pl.when(j > 0)
                    def _():
                        s_wait(pb)
                    g_start(i + 3, pb)
                else:
                    s_wait(pb)

                    @pl.when(j < NSW4 - 1)
                    def _():
                        g_start(i + 3, pb)
                if with_counts:
                    cnt_fire((j > 0) if b < 2 else True, i, b % 2)
            return 0
        lax.fori_loop(0, NSW4, body, 0)

        s_wait(3)
        if with_counts:
            @pl.when(c == 0)
            def _():
                pltpu.make_async_copy(ones, cnt_s.at[dstv.at[0]],
                                      cs[0]).wait()
                pltpu.make_async_copy(ones, cnt_s.at[dstv.at[0]],
                                      cs[1]).wait()

        plsc.subcore_barrier()
        pltpu.sync_copy(acc_s.at[pl.ds(base, RPT)],
                        out_hbm.at[c, pl.ds(base, RPT)])
        if with_counts:
            @pl.when(c == 0)
            def _():
                pltpu.sync_copy(cnt_s.at[pl.ds(base, RPT)],
                                outc_hbm.at[pl.ds(base, RPT)])

    return agg_kernel(x2, src3, src3p, dst3)


def _split_stack(h):
    return jnp.concatenate([h[:, :HD], h[:, HD:]], axis=0)


def _tc_self_path(x, W, b):
    def body(x_ref, W_ref, b_ref, o_ref):
        o_ref[...] = (jnp.dot(x_ref[...], W_ref[...],
                              preferred_element_type=jnp.float32)
                      + b_ref[...])

    return pl.pallas_call(
        body,
        out_shape=jax.ShapeDtypeStruct((N, D), jnp.float32),
    )(x, W, b)


def _tc_layer1(S, C, xr, Wl1, gamma, beta):
    def body(S_ref, C_ref, xr_ref, Wl1_ref, g_ref, b_ref,
             h2_ref, h2s_ref):
        inv = 1.0 / jnp.maximum(C_ref[:N, 0:1], 1.0)
        aggL = S_ref[0, :N, :] * inv
        aggR = S_ref[1, :N, :] * inv
        h = (jnp.dot(aggL, Wl1_ref[:HD, :],
                     preferred_element_type=jnp.float32)
             + jnp.dot(aggR, Wl1_ref[HD:, :],
                       preferred_element_type=jnp.float32)
             + xr_ref[...])
        mu = jnp.mean(h, axis=0, keepdims=True)
        var = jnp.mean((h - mu) * (h - mu), axis=0, keepdims=True)
        hn = (h - mu) / jnp.sqrt(var + 1e-5) * g_ref[...] + b_ref[...]
        h2 = jnp.where(hn >= 0, hn, 0.01 * hn)
        h2_ref[...] = h2
        h2s_ref[0] = h2[:, :HD]
        h2s_ref[1] = h2[:, HD:]

    return pl.pallas_call(
        body,
        out_shape=[
            jax.ShapeDtypeStruct((N, D), jnp.float32),
            jax.ShapeDtypeStruct((2, N, HD), jnp.float32),
        ],
    )(S, C, xr, Wl1, gamma, beta)


def _tc_layer2(S2, C, r2, Wl2):
    def body(S_ref, C_ref, r2_ref, Wl2_ref, out_ref):
        inv = 1.0 / jnp.maximum(C_ref[:N, 0:1], 1.0)
        aggL = S_ref[0, :N, :] * inv
        aggR = S_ref[1, :N, :] * inv
        out_ref[...] = (jnp.dot(aggL, Wl2_ref[:HD, :],
                                preferred_element_type=jnp.float32)
                        + jnp.dot(aggR, Wl2_ref[HD:, :],
                                  preferred_element_type=jnp.float32)
                        + r2_ref[...])

    return pl.pallas_call(
        body,
        out_shape=jax.ShapeDtypeStruct((N, D), jnp.float32),
    )(S2, C, r2, Wl2)


def kernel(x, edge_index, Wl1, bl1, Wr1, gamma, beta, Wl2, bl2, Wr2):
    src3 = edge_index[0].astype(jnp.int32).reshape(NS, NB, K)
    src3p = src3 + N
    dst3 = edge_index[1].astype(jnp.int32).reshape(NS, NB, K)
    bl1r = bl1.reshape(1, D)
    bl2r = bl2.reshape(1, D)
    gr = gamma.reshape(1, D)
    br = beta.reshape(1, D)

    S1, C = _sc_aggregate(_split_stack(x), src3, src3p, dst3,
                          with_counts=True)
    xr = _tc_self_path(x, Wr1, bl1r)
    h2, h2s = _tc_layer1(S1, C, xr, Wl1, gr, br)
    S2, _ = _sc_aggregate(h2s.reshape(2 * N, HD), src3, src3p, dst3,
                          with_counts=False)
    r2 = _tc_self_path(h2, Wr2, bl2r)
    return _tc_layer2(S2, C, r2, Wl2)
